# baseline (speedup 1.0000x reference)
import functools

import jax
import jax.numpy as jnp
import numpy as np
from jax import lax
from jax.experimental import pallas as pl
from jax.experimental.pallas import tpu as pltpu
from jax.experimental.pallas import tpu_sc as plsc

N = 10000
E = 320000
G = 64
NUM_NEIGHBORS = 32.0
INV_SQRT_NN = 1.0 / (NUM_NEIGHBORS ** 0.5)
DOUTS = [288, 288, 288, 64]

BE = 1280
BN = 1000
NSUB = 16
EPS = E // NSUB
C = 40
NCHUNK = EPS // C


def _ilv_ext(w):
    k, h = w.shape
    hb = 32 * ((h + 31) // 32)
    wp = jnp.pad(w, ((0, 0), (0, hb - h)))
    idx = np.empty((hb,), np.int32)
    for g in range(hb // 32):
        for i in range(16):
            idx[32 * g + 2 * i] = 32 * g + i
            idx[32 * g + 2 * i + 1] = 32 * g + 16 + i
    return wp[:, idx]


def _radial_body(elem_ref, ea_ref, f1, f2, f3a, f3b, wea, web, w_ref, p_ref):
    elem = elem_ref[...]
    ea = ea_ref[...]
    w2 = jax.nn.gelu(jnp.dot(elem, f1[...], preferred_element_type=jnp.float32))
    w2 = jax.nn.gelu(jnp.dot(w2, f2[...], preferred_element_type=jnp.float32))
    wa = jnp.dot(w2, f3a[...], preferred_element_type=jnp.float32)
    wb = jnp.dot(w2, f3b[...], preferred_element_type=jnp.float32)
    ema = jnp.dot(ea, wea[...], preferred_element_type=jnp.float32)
    emb = jnp.dot(ea, web[...], preferred_element_type=jnp.float32)
    w_ref[0] = wa.astype(jnp.bfloat16)
    w_ref[1] = wb.astype(jnp.bfloat16)
    p_ref[0] = (ema * wa).astype(jnp.bfloat16)
    p_ref[1] = (emb * wb).astype(jnp.bfloat16)


def _radial_call(elem, ea, wl):
    hb = wl[2].shape[1]
    full = lambda shape: pl.BlockSpec(shape, lambda e: tuple(0 for _ in shape))
    in_specs = [pl.BlockSpec((BE, 10), lambda e: (e, 0)),
                pl.BlockSpec((BE, 9), lambda e: (e, 0))] + \
               [full(w.shape) for w in wl]
    return pl.pallas_call(
        _radial_body,
        grid=(E // BE,),
        in_specs=in_specs,
        out_specs=[pl.BlockSpec((2, BE, hb), lambda e: (0, e, 0)),
                   pl.BlockSpec((2, BE, hb), lambda e: (0, e, 0))],
        out_shape=[jax.ShapeDtypeStruct((2, E, hb), jnp.bfloat16),
                   jax.ShapeDtypeStruct((2, E, hb), jnp.bfloat16)],
    )(elem, ea, *wl)


def _node0_body(x_ref, na_ref, wma, wmb, wsa, wsb, hm_ref, sc_ref):
    x = x_ref[...]
    na = na_ref[...]
    hm_ref[0] = jnp.dot(x, wma[...], preferred_element_type=jnp.float32)
    hm_ref[1] = jnp.dot(x, wmb[...], preferred_element_type=jnp.float32)
    sc_ref[0] = jnp.dot(x, wsa[...], preferred_element_type=jnp.float32) * na
    sc_ref[1] = jnp.dot(x, wsb[...], preferred_element_type=jnp.float32) * na


def _node0_call(x, na, wma, wmb, wsa, wsb):
    h = wma.shape[1]
    full = lambda shape: pl.BlockSpec(shape, lambda e: tuple(0 for _ in shape))
    return pl.pallas_call(
        _node0_body,
        grid=(N // BN,),
        in_specs=[pl.BlockSpec((BN, 128), lambda e: (e, 0)),
                  pl.BlockSpec((BN, 1), lambda e: (e, 0)),
                  full(wma.shape), full(wmb.shape), full(wsa.shape), full(wsb.shape)],
        out_specs=[pl.BlockSpec((2, BN, h), lambda e: (0, e, 0)),
                   pl.BlockSpec((2, BN, h), lambda e: (0, e, 0))],
        out_shape=[jax.ShapeDtypeStruct((2, N, h), jnp.float32),
                   jax.ShapeDtypeStruct((2, N, h), jnp.float32)],
    )(x, na, wma, wmb, wsa, wsb)


def _node_body(agg_ref, scp_ref, na_ref, wm00, wm01, wm10, wm11,
               ws00, ws01, ws10, ws11, hm_ref, sc_ref):
    h0 = jax.nn.gelu(agg_ref[0] * INV_SQRT_NN + scp_ref[0])
    h1 = jax.nn.gelu(agg_ref[1] * INV_SQRT_NN + scp_ref[1])
    na = na_ref[...]
    dot = lambda a, b: jnp.dot(a, b[...], preferred_element_type=jnp.float32)
    hm_ref[0] = dot(h0, wm00) + dot(h1, wm10)
    hm_ref[1] = dot(h0, wm01) + dot(h1, wm11)
    sc_ref[0] = (dot(h0, ws00) + dot(h1, ws10)) * na
    sc_ref[1] = (dot(h0, ws01) + dot(h1, ws11)) * na


def _node_call(agg, scp, na, wquads):
    hin = agg.shape[2]
    h = wquads[0].shape[1]
    full = lambda shape: pl.BlockSpec(shape, lambda e: tuple(0 for _ in shape))
    return pl.pallas_call(
        _node_body,
        grid=(N // BN,),
        in_specs=[pl.BlockSpec((2, BN, hin), lambda e: (0, e, 0)),
                  pl.BlockSpec((2, BN, hin), lambda e: (0, e, 0)),
                  pl.BlockSpec((BN, 1), lambda e: (e, 0))] +
                 [full(w.shape) for w in wquads],
        out_specs=[pl.BlockSpec((2, BN, h), lambda e: (0, e, 0)),
                   pl.BlockSpec((2, BN, h), lambda e: (0, e, 0))],
        out_shape=[jax.ShapeDtypeStruct((2, N, h), jnp.float32),
                   jax.ShapeDtypeStruct((2, N, h), jnp.float32)],
    )(agg, scp, na, *wquads)


def _make_sc_kernel(h):
    hb = 32 * ((h + 31) // 32)
    ngf = h // 32
    tail = (h % 32) != 0
    mesh = plsc.VectorSubcoreMesh(core_axis_name="c", subcore_axis_name="s")

    def body(hm_a, hm_b, w_hbm, p_hbm, src_hbm, dst_hbm, zeros_hbm, agg_hbm,
             spmem, rows0, rows1, w0, w1, p0, p1,
             srcv0, srcv1, srcv2, srcv3, dstv0, dstv1, dstv2, dstv3,
             sem_g0, sem_g1, sem_w0, sem_w1, sem_s0, sem_s1,
             sem_i0, sem_i1, sem_i2, sem_i3):
        c = lax.axis_index("c")
        s = lax.axis_index("s")
        rowsb = (rows0, rows1)
        wb = (w0, w1)
        pb = (p0, p1)
        srcv = (srcv0, srcv1, srcv2, srcv3)
        dstv = (dstv0, dstv1, dstv2, dstv3)
        sem_g = (sem_g0, sem_g1)
        sem_w = (sem_w0, sem_w1)
        sem_s = (sem_s0, sem_s1)
        sem_i = (sem_i0, sem_i1, sem_i2, sem_i3)

        @pl.when(s < 10)
        def _():
            pltpu.sync_copy(zeros_hbm, spmem.at[pl.ds(s * 1000, 1000)])

        base0 = s * EPS

        def issue_idx(k, sl):
            pltpu.async_copy(src_hbm.at[pl.ds(base0 + k * C, C)], srcv[sl],
                             sem_i[sl])
            pltpu.async_copy(dst_hbm.at[pl.ds(base0 + k * C, C)], dstv[sl],
                             sem_i[sl])

        def wait_idx(sl):
            pltpu.make_async_copy(src_hbm.at[pl.ds(0, C)], srcv[sl],
                                  sem_i[sl]).wait()
            pltpu.make_async_copy(src_hbm.at[pl.ds(0, C)], dstv[sl],
                                  sem_i[sl]).wait()

        def run(hm_hbm, ebase0):
            def issue(k, sl, q):
                pltpu.async_copy(hm_hbm.at[srcv[sl]], rowsb[q], sem_g[q])
                base = ebase0 + base0 + k * C
                pltpu.async_copy(w_hbm.at[pl.ds(base, C)], wb[q], sem_w[q])
                pltpu.async_copy(p_hbm.at[pl.ds(base, C)], pb[q], sem_w[q])

            issue_idx(0, 0)
            issue_idx(1, 1)
            issue_idx(2, 2)
            wait_idx(0)
            issue(0, 0, 0)

            def chunk(k, sl, p, q):
                pltpu.make_async_copy(hm_hbm.at[srcv[sl]], rowsb[p],
                                      sem_g[p]).wait()
                pltpu.make_async_copy(w_hbm.at[pl.ds(0, C)], wb[p],
                                      sem_w[p]).wait()
                pltpu.make_async_copy(w_hbm.at[pl.ds(0, C)], pb[p],
                                      sem_w[p]).wait()

                def edge(i, carry):
                    for t in range(ngf + (1 if tail else 0)):
                        gsl = pl.ds(32 * t, 32)
                        wab = plsc.unpack(wb[p][i, gsl],
                                          format=plsc.PackFormat.INTERLEAVED,
                                          preferred_element_type=jnp.float32)
                        pab = plsc.unpack(pb[p][i, gsl],
                                          format=plsc.PackFormat.INTERLEAVED,
                                          preferred_element_type=jnp.float32)
                        sl0 = pl.ds(32 * t, 16)
                        rowsb[p][i, sl0] = (rowsb[p][i, sl0] * wab[0] + pab[0])
                        if t < ngf:
                            sl1 = pl.ds(32 * t + 16, 16)
                            rowsb[p][i, sl1] = (rowsb[p][i, sl1] * wab[1]
                                                + pab[1])
                    return carry
                lax.fori_loop(0, C, edge, 0, unroll=2)

                pltpu.async_copy(rowsb[p], spmem.at[dstv[sl]], sem_s[p],
                                 add=True)

                @pl.when(k + 1 < NCHUNK)
                def _():
                    @pl.when(k >= 1)
                    def _():
                        pltpu.make_async_copy(w_hbm.at[pl.ds(0, C)], rowsb[q],
                                              sem_s[q]).wait()
                    wait_idx((sl + 1) % 4)
                    issue(k + 1, (sl + 1) % 4, q)

                    @pl.when(k + 3 < NCHUNK)
                    def _():
                        issue_idx(k + 3, (sl + 3) % 4)

            def quad(jj, carry):
                for u in range(4):
                    chunk(4 * jj + u, u, u % 2, 1 - u % 2)
                return carry
            lax.fori_loop(0, NCHUNK // 4, quad, 0)

            pltpu.make_async_copy(w_hbm.at[pl.ds(0, C)], rows0, sem_s0).wait()
            pltpu.make_async_copy(w_hbm.at[pl.ds(0, C)], rows1, sem_s1).wait()

        @pl.when(c == 0)
        def _():
            run(hm_a, 0)

        @pl.when(c == 1)
        def _():
            run(hm_b, E)

        plsc.subcore_barrier()

        @pl.when(s < 10)
        def _():
            pltpu.sync_copy(spmem.at[pl.ds(s * 1000, 1000)],
                            agg_hbm.at[pl.ds(c * N + s * 1000, 1000)])

    return pl.kernel(
        body,
        out_type=jax.ShapeDtypeStruct((2 * N, h), jnp.float32),
        mesh=mesh,
        compiler_params=pltpu.CompilerParams(use_tc_tiling_on_sc=False,
                                             needs_layout_passes=False),
        scratch_types=[
            pltpu.VMEM_SHARED((N, h), jnp.float32),
            pltpu.VMEM((C, h), jnp.float32),
            pltpu.VMEM((C, h), jnp.float32),
            pltpu.VMEM((C, hb), jnp.bfloat16),
            pltpu.VMEM((C, hb), jnp.bfloat16),
            pltpu.VMEM((C, hb), jnp.bfloat16),
            pltpu.VMEM((C, hb), jnp.bfloat16),
            pltpu.VMEM((C,), jnp.int32),
            pltpu.VMEM((C,), jnp.int32),
            pltpu.VMEM((C,), jnp.int32),
            pltpu.VMEM((C,), jnp.int32),
            pltpu.VMEM((C,), jnp.int32),
            pltpu.VMEM((C,), jnp.int32),
            pltpu.VMEM((C,), jnp.int32),
            pltpu.VMEM((C,), jnp.int32),
        ] + [pltpu.SemaphoreType.DMA] * 10,
    )


def _final_body(agg_ref, scp_ref, batch_ref, fc1w, fc1b, fc2w, fc2b, ow, ob,
                out_ref):
    h0 = agg_ref[0] * INV_SQRT_NN + scp_ref[0]
    h1 = agg_ref[1] * INV_SQRT_NN + scp_ref[1]
    seg = lax.broadcasted_iota(jnp.int32, (G, N), 0)
    mask = (batch_ref[...] == seg).astype(jnp.float32)
    g0 = jnp.dot(mask, h0, preferred_element_type=jnp.float32)
    g1 = jnp.dot(mask, h1, preferred_element_type=jnp.float32)
    g = jnp.concatenate([g0, g1], axis=1)
    g = jax.nn.gelu(jnp.dot(g, fc1w[...], preferred_element_type=jnp.float32) + fc1b[...])
    g = jax.nn.gelu(jnp.dot(g, fc2w[...], preferred_element_type=jnp.float32) + fc2b[...])
    logits = jnp.dot(g, ow[...], preferred_element_type=jnp.float32) + ob[...]
    out_ref[...] = jax.nn.softmax(logits, axis=1)


def _final_call(agg, scp, batch2d, fc1w, fc1b, fc2w, fc2b, ow, ob):
    full = lambda shape: pl.BlockSpec(shape, lambda: tuple(0 for _ in shape))
    args = (agg, scp, batch2d, fc1w, fc1b, fc2w, fc2b, ow, ob)
    return pl.pallas_call(
        _final_body,
        in_specs=[full(a.shape) for a in args],
        out_specs=full((G, 16)),
        out_shape=jax.ShapeDtypeStruct((G, 16), jnp.float32),
    )(*args)


def kernel(x, node_attr, edge_src, edge_dst, edge_attr, edge_length_embedding,
           batch, params):
    halves = [d // 2 for d in DOUTS]

    na = node_attr
    src = edge_src.astype(jnp.int32)
    dst = edge_dst.astype(jnp.int32)

    wps = []
    for l in range(4):
        h = halves[l]
        f3 = params[f'F3_{l}']
        we = params[f'Wedge{l}']
        wl = [params[f'F1_{l}'], params[f'F2_{l}'],
              _ilv_ext(f3[:, :h]), _ilv_ext(f3[:, h:]),
              _ilv_ext(we[:, :h]), _ilv_ext(we[:, h:])]
        wps.append(_radial_call(edge_length_embedding, edge_attr, wl))

    wm, ws = params['Wmsg0'], params['Wsc0']
    h0 = halves[0]
    hm, sc = _node0_call(x, na, wm[:, :h0], wm[:, h0:], ws[:, :h0], ws[:, h0:])

    agg = None
    for l in range(4):
        h = halves[l]
        w_st, p_st = wps[l]
        hb = 32 * ((h + 31) // 32)
        zeros = jnp.zeros((1000, h), jnp.float32)
        sck = _make_sc_kernel(h)
        agg = sck(hm[0], hm[1], w_st.reshape(2 * E, hb),
                  p_st.reshape(2 * E, hb), src, dst, zeros)
        if l < 3:
            hn = halves[l + 1]
            hin = h
            wm, ws = params[f'Wmsg{l + 1}'], params[f'Wsc{l + 1}']
            quads = [wm[:hin, :hn], wm[:hin, hn:], wm[hin:, :hn], wm[hin:, hn:],
                     ws[:hin, :hn], ws[:hin, hn:], ws[hin:, :hn], ws[hin:, hn:]]
            hm, sc = _node_call(agg.reshape(2, N, h), sc, na, quads)

    p = params
    return _final_call(agg.reshape(2, N, halves[3]), sc, batch.reshape(1, N),
                       p['fc1_w'], p['fc1_b'].reshape(1, -1),
                       p['fc2_w'], p['fc2_b'].reshape(1, -1),
                       p['out_w'], p['out_b'].reshape(1, -1))


# speedup vs baseline: 1.2486x; 1.2486x over previous
import functools

import jax
import jax.numpy as jnp
import numpy as np
from jax import lax
from jax.experimental import pallas as pl
from jax.experimental.pallas import tpu as pltpu
from jax.experimental.pallas import tpu_sc as plsc

N = 10000
E = 320000
G = 64
NUM_NEIGHBORS = 32.0
INV_SQRT_NN = 1.0 / (NUM_NEIGHBORS ** 0.5)
DOUTS = [288, 288, 288, 64]

BE = 1280
BN = 1000
NSUB = 16
EPS = E // NSUB
C = 40
CG = C // 8
NCHUNK = EPS // C


def _radial_body144(elem_ref, ea_ref, f1, f2, f3a, f3b, wea, web, w_ref, p_ref):
    elem = elem_ref[...]
    ea = ea_ref[...]
    w2 = jax.nn.gelu(jnp.dot(elem, f1[...], preferred_element_type=jnp.float32))
    w2 = jax.nn.gelu(jnp.dot(w2, f2[...], preferred_element_type=jnp.float32))
    wa = jnp.dot(w2, f3a[...], preferred_element_type=jnp.float32)
    wb = jnp.dot(w2, f3b[...], preferred_element_type=jnp.float32)
    ema = jnp.dot(ea, wea[...], preferred_element_type=jnp.float32)
    emb = jnp.dot(ea, web[...], preferred_element_type=jnp.float32)
    pa = ema * wa
    pb = emb * wb
    ng = BE // 8
    for ref, va, vb in ((w_ref, wa, wb), (p_ref, pa, pb)):
        ref[0, :, 0] = va[:, :128].reshape(ng, 8, 128)
        ref[0, :, 1, :, :16] = va[:, 128:].reshape(ng, 8, 16)
        ref[1, :, 0] = vb[:, :128].reshape(ng, 8, 128)
        ref[1, :, 1, :, :16] = vb[:, 128:].reshape(ng, 8, 16)


def _radial_call144(elem, ea, wl):
    full = lambda shape: pl.BlockSpec(shape, lambda e: tuple(0 for _ in shape))
    in_specs = [pl.BlockSpec((BE, 10), lambda e: (e, 0)),
                pl.BlockSpec((BE, 9), lambda e: (e, 0))] + \
               [full(w.shape) for w in wl]
    ospec = pl.BlockSpec((2, BE // 8, 2, 8, 128), lambda e: (0, e, 0, 0, 0))
    oshape = jax.ShapeDtypeStruct((2, E // 8, 2, 8, 128), jnp.float32)
    return pl.pallas_call(
        _radial_body144,
        grid=(E // BE,),
        in_specs=in_specs,
        out_specs=[ospec, ospec],
        out_shape=[oshape, oshape],
    )(elem, ea, *wl)


def _radial_body32(elem_ref, ea_ref, f1, f2, f3a, f3b, wea, web, w_ref, p_ref):
    elem = elem_ref[...]
    ea = ea_ref[...]
    w2 = jax.nn.gelu(jnp.dot(elem, f1[...], preferred_element_type=jnp.float32))
    w2 = jax.nn.gelu(jnp.dot(w2, f2[...], preferred_element_type=jnp.float32))
    wa = jnp.dot(w2, f3a[...], preferred_element_type=jnp.float32)
    wb = jnp.dot(w2, f3b[...], preferred_element_type=jnp.float32)
    ema = jnp.dot(ea, wea[...], preferred_element_type=jnp.float32)
    emb = jnp.dot(ea, web[...], preferred_element_type=jnp.float32)
    w_ref[0] = wa
    w_ref[1] = wb
    p_ref[0] = ema * wa
    p_ref[1] = emb * wb


def _radial_call32(elem, ea, wl):
    h = wl[2].shape[1]
    full = lambda shape: pl.BlockSpec(shape, lambda e: tuple(0 for _ in shape))
    in_specs = [pl.BlockSpec((BE, 10), lambda e: (e, 0)),
                pl.BlockSpec((BE, 9), lambda e: (e, 0))] + \
               [full(w.shape) for w in wl]
    return pl.pallas_call(
        _radial_body32,
        grid=(E // BE,),
        in_specs=in_specs,
        out_specs=[pl.BlockSpec((2, BE, h), lambda e: (0, e, 0)),
                   pl.BlockSpec((2, BE, h), lambda e: (0, e, 0))],
        out_shape=[jax.ShapeDtypeStruct((2, E, h), jnp.float32),
                   jax.ShapeDtypeStruct((2, E, h), jnp.float32)],
    )(elem, ea, *wl)


def _node0_body(x_ref, na_ref, wma, wmb, wsa, wsb, hm_ref, sc_ref):
    x = x_ref[...]
    na = na_ref[...]
    hm_ref[0] = jnp.dot(x, wma[...], preferred_element_type=jnp.float32)
    hm_ref[1] = jnp.dot(x, wmb[...], preferred_element_type=jnp.float32)
    sc_ref[0] = jnp.dot(x, wsa[...], preferred_element_type=jnp.float32) * na
    sc_ref[1] = jnp.dot(x, wsb[...], preferred_element_type=jnp.float32) * na


def _node0_call(x, na, wma, wmb, wsa, wsb):
    h = wma.shape[1]
    full = lambda shape: pl.BlockSpec(shape, lambda e: tuple(0 for _ in shape))
    return pl.pallas_call(
        _node0_body,
        grid=(N // BN,),
        in_specs=[pl.BlockSpec((BN, 128), lambda e: (e, 0)),
                  pl.BlockSpec((BN, 1), lambda e: (e, 0)),
                  full(wma.shape), full(wmb.shape), full(wsa.shape), full(wsb.shape)],
        out_specs=[pl.BlockSpec((2, BN, h), lambda e: (0, e, 0)),
                   pl.BlockSpec((2, BN, h), lambda e: (0, e, 0))],
        out_shape=[jax.ShapeDtypeStruct((2, N, h), jnp.float32),
                   jax.ShapeDtypeStruct((2, N, h), jnp.float32)],
    )(x, na, wma, wmb, wsa, wsb)


def _node_body(agg_ref, scp_ref, na_ref, wm00, wm01, wm10, wm11,
               ws00, ws01, ws10, ws11, hm_ref, sc_ref):
    h0 = jax.nn.gelu(agg_ref[0] * INV_SQRT_NN + scp_ref[0])
    h1 = jax.nn.gelu(agg_ref[1] * INV_SQRT_NN + scp_ref[1])
    na = na_ref[...]
    dot = lambda a, b: jnp.dot(a, b[...], preferred_element_type=jnp.float32)
    hm_ref[0] = dot(h0, wm00) + dot(h1, wm10)
    hm_ref[1] = dot(h0, wm01) + dot(h1, wm11)
    sc_ref[0] = (dot(h0, ws00) + dot(h1, ws10)) * na
    sc_ref[1] = (dot(h0, ws01) + dot(h1, ws11)) * na


def _node_call(agg, scp, na, wquads):
    hin = agg.shape[2]
    h = wquads[0].shape[1]
    full = lambda shape: pl.BlockSpec(shape, lambda e: tuple(0 for _ in shape))
    return pl.pallas_call(
        _node_body,
        grid=(N // BN,),
        in_specs=[pl.BlockSpec((2, BN, hin), lambda e: (0, e, 0)),
                  pl.BlockSpec((2, BN, hin), lambda e: (0, e, 0)),
                  pl.BlockSpec((BN, 1), lambda e: (e, 0))] +
                 [full(w.shape) for w in wquads],
        out_specs=[pl.BlockSpec((2, BN, h), lambda e: (0, e, 0)),
                   pl.BlockSpec((2, BN, h), lambda e: (0, e, 0))],
        out_shape=[jax.ShapeDtypeStruct((2, N, h), jnp.float32),
                   jax.ShapeDtypeStruct((2, N, h), jnp.float32)],
    )(agg, scp, na, *wquads)


def _make_sc_kernel(h):
    tiledwp = (h == 144)
    nv = h // 16
    mesh = plsc.VectorSubcoreMesh(core_axis_name="c", subcore_axis_name="s")

    if tiledwp:
        wp_scratch = [pltpu.VMEM((CG, 8, 128), jnp.float32),
                      pltpu.VMEM((CG, 8, 128), jnp.float32),
                      pltpu.VMEM((CG, 8, 16), jnp.float32),
                      pltpu.VMEM((CG, 8, 16), jnp.float32),
                      pltpu.VMEM((CG, 8, 128), jnp.float32),
                      pltpu.VMEM((CG, 8, 128), jnp.float32),
                      pltpu.VMEM((CG, 8, 16), jnp.float32),
                      pltpu.VMEM((CG, 8, 16), jnp.float32)]
    else:
        wp_scratch = [pltpu.VMEM((C, h), jnp.float32),
                      pltpu.VMEM((C, h), jnp.float32),
                      pltpu.VMEM((C, h), jnp.float32),
                      pltpu.VMEM((C, h), jnp.float32)]

    def body(hm_a, hm_b, w_hbm, p_hbm, src_hbm, dst_hbm, zeros_hbm, agg_hbm,
             spmem, rows0, rows1, *rest):
        wpb = rest[:len(wp_scratch)]
        rest = rest[len(wp_scratch):]
        (srcv0, srcv1, srcv2, srcv3, dstv0, dstv1, dstv2, dstv3,
         sem_g0, sem_g1, sem_w0, sem_w1, sem_s0, sem_s1,
         sem_i0, sem_i1, sem_i2, sem_i3) = rest
        c = lax.axis_index("c")
        s = lax.axis_index("s")
        rowsb = (rows0, rows1)
        if tiledwp:
            wbig = (wpb[0], wpb[1])
            wsm = (wpb[2], wpb[3])
            pbig = (wpb[4], wpb[5])
            psm = (wpb[6], wpb[7])
        else:
            wb = (wpb[0], wpb[1])
            pb = (wpb[2], wpb[3])
        srcv = (srcv0, srcv1, srcv2, srcv3)
        dstv = (dstv0, dstv1, dstv2, dstv3)
        sem_g = (sem_g0, sem_g1)
        sem_w = (sem_w0, sem_w1)
        sem_s = (sem_s0, sem_s1)
        sem_i = (sem_i0, sem_i1, sem_i2, sem_i3)

        @pl.when(s < 10)
        def _():
            pltpu.sync_copy(zeros_hbm, spmem.at[pl.ds(s * 1000, 1000)])

        base0 = s * EPS

        def issue_idx(k, sl):
            pltpu.async_copy(src_hbm.at[pl.ds(base0 + k * C, C)], srcv[sl],
                             sem_i[sl])
            pltpu.async_copy(dst_hbm.at[pl.ds(base0 + k * C, C)], dstv[sl],
                             sem_i[sl])

        def wait_idx(sl):
            pltpu.make_async_copy(src_hbm.at[pl.ds(0, C)], srcv[sl],
                                  sem_i[sl]).wait()
            pltpu.make_async_copy(src_hbm.at[pl.ds(0, C)], dstv[sl],
                                  sem_i[sl]).wait()

        def run(hm_hbm, core):
            def issue(k, sl, q):
                pltpu.async_copy(hm_hbm.at[srcv[sl]], rowsb[q], sem_g[q])
                if tiledwp:
                    gb = core * (E // 8) + (base0 + k * C) // 8
                    pltpu.async_copy(w_hbm.at[pl.ds(gb, CG), 0], wbig[q],
                                     sem_w[q])
                    pltpu.async_copy(w_hbm.at[pl.ds(gb, CG), 1, :, pl.ds(0, 16)],
                                     wsm[q], sem_w[q])
                    pltpu.async_copy(p_hbm.at[pl.ds(gb, CG), 0], pbig[q],
                                     sem_w[q])
                    pltpu.async_copy(p_hbm.at[pl.ds(gb, CG), 1, :, pl.ds(0, 16)],
                                     psm[q], sem_w[q])
                else:
                    base = core * E + base0 + k * C
                    pltpu.async_copy(w_hbm.at[pl.ds(base, C)], wpb[0 + q],
                                     sem_w[q])
                    pltpu.async_copy(p_hbm.at[pl.ds(base, C)], wpb[2 + q],
                                     sem_w[q])

            def wait_wp(p):
                if tiledwp:
                    pltpu.make_async_copy(w_hbm.at[pl.ds(0, CG), 0],
                                          wbig[p], sem_w[p]).wait()
                    pltpu.make_async_copy(w_hbm.at[pl.ds(0, CG), 1, :,
                                                   pl.ds(0, 16)],
                                          wsm[p], sem_w[p]).wait()
                    pltpu.make_async_copy(p_hbm.at[pl.ds(0, CG), 0],
                                          pbig[p], sem_w[p]).wait()
                    pltpu.make_async_copy(p_hbm.at[pl.ds(0, CG), 1, :,
                                                   pl.ds(0, 16)],
                                          psm[p], sem_w[p]).wait()
                else:
                    pltpu.make_async_copy(w_hbm.at[pl.ds(0, C)],
                                          wpb[0 + p], sem_w[p]).wait()
                    pltpu.make_async_copy(p_hbm.at[pl.ds(0, C)],
                                          wpb[2 + p], sem_w[p]).wait()

            issue_idx(0, 0)
            issue_idx(1, 1)
            issue_idx(2, 2)
            wait_idx(0)
            issue(0, 0, 0)

            def chunk(k, sl, p, q):
                pltpu.make_async_copy(hm_hbm.at[srcv[sl]], rowsb[p],
                                      sem_g[p]).wait()
                wait_wp(p)

                def edge(i, carry):
                    if tiledwp:
                        g = i // 8
                        r = i % 8
                        for t in range(8):
                            esl = pl.ds(t * 16, 16)
                            rowsb[p][i, esl] = (rowsb[p][i, esl]
                                                * wbig[p][g, r, esl]
                                                + pbig[p][g, r, esl])
                        esl = pl.ds(128, 16)
                        rowsb[p][i, esl] = (rowsb[p][i, esl]
                                            * wsm[p][g, r, :]
                                            + psm[p][g, r, :])
                    else:
                        for t in range(nv):
                            esl = pl.ds(t * 16, 16)
                            rowsb[p][i, esl] = (rowsb[p][i, esl]
                                                * wpb[0 + p][i, esl]
                                                + wpb[2 + p][i, esl])
                    return carry
                lax.fori_loop(0, C, edge, 0, unroll=2)

                pltpu.async_copy(rowsb[p], spmem.at[dstv[sl]], sem_s[p],
                                 add=True)

                @pl.when(k + 1 < NCHUNK)
                def _():
                    @pl.when(k >= 1)
                    def _():
                        pltpu.make_async_copy(hm_a.at[pl.ds(0, C)], rowsb[q],
                                              sem_s[q]).wait()
                    wait_idx((sl + 1) % 4)
                    issue(k + 1, (sl + 1) % 4, q)

                    @pl.when(k + 3 < NCHUNK)
                    def _():
                        issue_idx(k + 3, (sl + 3) % 4)

            def quad(jj, carry):
                for u in range(4):
                    chunk(4 * jj + u, u, u % 2, 1 - u % 2)
                return carry
            lax.fori_loop(0, NCHUNK // 4, quad, 0)

            pltpu.make_async_copy(hm_a.at[pl.ds(0, C)], rows0, sem_s0).wait()
            pltpu.make_async_copy(hm_a.at[pl.ds(0, C)], rows1, sem_s1).wait()

        @pl.when(c == 0)
        def _():
            run(hm_a, 0)

        @pl.when(c == 1)
        def _():
            run(hm_b, 1)

        plsc.subcore_barrier()

        @pl.when(s < 10)
        def _():
            pltpu.sync_copy(spmem.at[pl.ds(s * 1000, 1000)],
                            agg_hbm.at[pl.ds(c * N + s * 1000, 1000)])

    return pl.kernel(
        body,
        out_type=jax.ShapeDtypeStruct((2 * N, h), jnp.float32),
        mesh=mesh,
        compiler_params=pltpu.CompilerParams(use_tc_tiling_on_sc=False,
                                             needs_layout_passes=False),
        scratch_types=[
            pltpu.VMEM_SHARED((N, h), jnp.float32),
            pltpu.VMEM((C, h), jnp.float32),
            pltpu.VMEM((C, h), jnp.float32),
        ] + wp_scratch + [
            pltpu.VMEM((C,), jnp.int32),
            pltpu.VMEM((C,), jnp.int32),
            pltpu.VMEM((C,), jnp.int32),
            pltpu.VMEM((C,), jnp.int32),
            pltpu.VMEM((C,), jnp.int32),
            pltpu.VMEM((C,), jnp.int32),
            pltpu.VMEM((C,), jnp.int32),
            pltpu.VMEM((C,), jnp.int32),
        ] + [pltpu.SemaphoreType.DMA] * 10,
    )


def _final_body(agg_ref, scp_ref, batch_ref, fc1w, fc1b, fc2w, fc2b, ow, ob,
                out_ref):
    h0 = agg_ref[0] * INV_SQRT_NN + scp_ref[0]
    h1 = agg_ref[1] * INV_SQRT_NN + scp_ref[1]
    seg = lax.broadcasted_iota(jnp.int32, (G, N), 0)
    mask = (batch_ref[...] == seg).astype(jnp.float32)
    g0 = jnp.dot(mask, h0, preferred_element_type=jnp.float32)
    g1 = jnp.dot(mask, h1, preferred_element_type=jnp.float32)
    g = jnp.concatenate([g0, g1], axis=1)
    g = jax.nn.gelu(jnp.dot(g, fc1w[...], preferred_element_type=jnp.float32) + fc1b[...])
    g = jax.nn.gelu(jnp.dot(g, fc2w[...], preferred_element_type=jnp.float32) + fc2b[...])
    logits = jnp.dot(g, ow[...], preferred_element_type=jnp.float32) + ob[...]
    out_ref[...] = jax.nn.softmax(logits, axis=1)


def _final_call(agg, scp, batch2d, fc1w, fc1b, fc2w, fc2b, ow, ob):
    full = lambda shape: pl.BlockSpec(shape, lambda: tuple(0 for _ in shape))
    args = (agg, scp, batch2d, fc1w, fc1b, fc2w, fc2b, ow, ob)
    return pl.pallas_call(
        _final_body,
        in_specs=[full(a.shape) for a in args],
        out_specs=full((G, 16)),
        out_shape=jax.ShapeDtypeStruct((G, 16), jnp.float32),
    )(*args)


def kernel(x, node_attr, edge_src, edge_dst, edge_attr, edge_length_embedding,
           batch, params):
    halves = [d // 2 for d in DOUTS]

    na = node_attr
    src = edge_src.astype(jnp.int32)
    dst = edge_dst.astype(jnp.int32)

    wps = []
    for l in range(4):
        h = halves[l]
        f3 = params[f'F3_{l}']
        we = params[f'Wedge{l}']
        wl = [params[f'F1_{l}'], params[f'F2_{l}'],
              f3[:, :h], f3[:, h:], we[:, :h], we[:, h:]]
        if h == 144:
            wps.append(_radial_call144(edge_length_embedding, edge_attr, wl))
        else:
            wps.append(_radial_call32(edge_length_embedding, edge_attr, wl))

    wm, ws = params['Wmsg0'], params['Wsc0']
    h0 = halves[0]
    hm, sc = _node0_call(x, na, wm[:, :h0], wm[:, h0:], ws[:, :h0], ws[:, h0:])

    agg = None
    for l in range(4):
        h = halves[l]
        w_st, p_st = wps[l]
        if h == 144:
            w_in = w_st.reshape(2 * (E // 8), 2, 8, 128)
            p_in = p_st.reshape(2 * (E // 8), 2, 8, 128)
        else:
            w_in = w_st.reshape(2 * E, h)
            p_in = p_st.reshape(2 * E, h)
        zeros = jnp.zeros((1000, h), jnp.float32)
        sck = _make_sc_kernel(h)
        agg = sck(hm[0], hm[1], w_in, p_in, src, dst, zeros)
        if l < 3:
            hn = halves[l + 1]
            hin = h
            wm, ws = params[f'Wmsg{l + 1}'], params[f'Wsc{l + 1}']
            quads = [wm[:hin, :hn], wm[:hin, hn:], wm[hin:, :hn], wm[hin:, hn:],
                     ws[:hin, :hn], ws[:hin, hn:], ws[hin:, :hn], ws[hin:, hn:]]
            hm, sc = _node_call(agg.reshape(2, N, h), sc, na, quads)

    p = params
    return _final_call(agg.reshape(2, N, halves[3]), sc, batch.reshape(1, N),
                       p['fc1_w'], p['fc1_b'].reshape(1, -1),
                       p['fc2_w'], p['fc2_b'].reshape(1, -1),
                       p['out_w'], p['out_b'].reshape(1, -1))


# speedup vs baseline: 1.6652x; 1.3336x over previous
import functools

import jax
import jax.numpy as jnp
import numpy as np
from jax import lax
from jax.experimental import pallas as pl
from jax.experimental.pallas import tpu as pltpu
from jax.experimental.pallas import tpu_sc as plsc

N = 10000
E = 320000
G = 64
NUM_NEIGHBORS = 32.0
INV_SQRT_NN = 1.0 / (NUM_NEIGHBORS ** 0.5)
DOUTS = [288, 288, 288, 64]

BE = 1280
BN = 1000
NSUB = 16
EPS = E // NSUB
C = 32
CG = C // 8
NCHUNK = EPS // C


def _radial_body144(elem_ref, ea_ref, f1, f2, f3a, f3b, wea, web, w_ref, p_ref):
    elem = elem_ref[...]
    ea = ea_ref[...]
    w2 = jax.nn.gelu(jnp.dot(elem, f1[...], preferred_element_type=jnp.float32))
    w2 = jax.nn.gelu(jnp.dot(w2, f2[...], preferred_element_type=jnp.float32))
    wa = jnp.dot(w2, f3a[...], preferred_element_type=jnp.float32)
    wb = jnp.dot(w2, f3b[...], preferred_element_type=jnp.float32)
    ema = jnp.dot(ea, wea[...], preferred_element_type=jnp.float32)
    emb = jnp.dot(ea, web[...], preferred_element_type=jnp.float32)
    pa = ema * wa
    pb = emb * wb
    ng = BE // 8
    for ref, va, vb in ((w_ref, wa, wb), (p_ref, pa, pb)):
        ref[0, :, 0] = va[:, :128].reshape(ng, 8, 128)
        ref[0, :, 1, :, :16] = va[:, 128:].reshape(ng, 8, 16)
        ref[1, :, 0] = vb[:, :128].reshape(ng, 8, 128)
        ref[1, :, 1, :, :16] = vb[:, 128:].reshape(ng, 8, 16)


def _radial_call144(elem, ea, wl):
    full = lambda shape: pl.BlockSpec(shape, lambda e: tuple(0 for _ in shape))
    in_specs = [pl.BlockSpec((BE, 10), lambda e: (e, 0)),
                pl.BlockSpec((BE, 9), lambda e: (e, 0))] + \
               [full(w.shape) for w in wl]
    ospec = pl.BlockSpec((2, BE // 8, 2, 8, 128), lambda e: (0, e, 0, 0, 0))
    oshape = jax.ShapeDtypeStruct((2, E // 8, 2, 8, 128), jnp.float32)
    return pl.pallas_call(
        _radial_body144,
        grid=(E // BE,),
        in_specs=in_specs,
        out_specs=[ospec, ospec],
        out_shape=[oshape, oshape],
    )(elem, ea, *wl)


def _radial_body32(elem_ref, ea_ref, f1, f2, f3a, f3b, wea, web, w_ref, p_ref):
    elem = elem_ref[...]
    ea = ea_ref[...]
    w2 = jax.nn.gelu(jnp.dot(elem, f1[...], preferred_element_type=jnp.float32))
    w2 = jax.nn.gelu(jnp.dot(w2, f2[...], preferred_element_type=jnp.float32))
    wa = jnp.dot(w2, f3a[...], preferred_element_type=jnp.float32)
    wb = jnp.dot(w2, f3b[...], preferred_element_type=jnp.float32)
    ema = jnp.dot(ea, wea[...], preferred_element_type=jnp.float32)
    emb = jnp.dot(ea, web[...], preferred_element_type=jnp.float32)
    w_ref[0] = wa
    w_ref[1] = wb
    p_ref[0] = ema * wa
    p_ref[1] = emb * wb


def _radial_call32(elem, ea, wl):
    h = wl[2].shape[1]
    full = lambda shape: pl.BlockSpec(shape, lambda e: tuple(0 for _ in shape))
    in_specs = [pl.BlockSpec((BE, 10), lambda e: (e, 0)),
                pl.BlockSpec((BE, 9), lambda e: (e, 0))] + \
               [full(w.shape) for w in wl]
    return pl.pallas_call(
        _radial_body32,
        grid=(E // BE,),
        in_specs=in_specs,
        out_specs=[pl.BlockSpec((2, BE, h), lambda e: (0, e, 0)),
                   pl.BlockSpec((2, BE, h), lambda e: (0, e, 0))],
        out_shape=[jax.ShapeDtypeStruct((2, E, h), jnp.float32),
                   jax.ShapeDtypeStruct((2, E, h), jnp.float32)],
    )(elem, ea, *wl)


def _node0_body(x_ref, na_ref, wma, wmb, wsa, wsb, hm_ref, sc_ref):
    x = x_ref[...]
    na = na_ref[...]
    hm_ref[0] = jnp.dot(x, wma[...], preferred_element_type=jnp.float32)
    hm_ref[1] = jnp.dot(x, wmb[...], preferred_element_type=jnp.float32)
    sc_ref[0] = jnp.dot(x, wsa[...], preferred_element_type=jnp.float32) * na
    sc_ref[1] = jnp.dot(x, wsb[...], preferred_element_type=jnp.float32) * na


def _node0_call(x, na, wma, wmb, wsa, wsb):
    h = wma.shape[1]
    full = lambda shape: pl.BlockSpec(shape, lambda e: tuple(0 for _ in shape))
    return pl.pallas_call(
        _node0_body,
        grid=(N // BN,),
        in_specs=[pl.BlockSpec((BN, 128), lambda e: (e, 0)),
                  pl.BlockSpec((BN, 1), lambda e: (e, 0)),
                  full(wma.shape), full(wmb.shape), full(wsa.shape), full(wsb.shape)],
        out_specs=[pl.BlockSpec((2, BN, h), lambda e: (0, e, 0)),
                   pl.BlockSpec((2, BN, h), lambda e: (0, e, 0))],
        out_shape=[jax.ShapeDtypeStruct((2, N, h), jnp.float32),
                   jax.ShapeDtypeStruct((2, N, h), jnp.float32)],
    )(x, na, wma, wmb, wsa, wsb)


def _node_body(agg_ref, scp_ref, na_ref, wm00, wm01, wm10, wm11,
               ws00, ws01, ws10, ws11, hm_ref, sc_ref):
    h0 = jax.nn.gelu(agg_ref[0] * INV_SQRT_NN + scp_ref[0])
    h1 = jax.nn.gelu(agg_ref[1] * INV_SQRT_NN + scp_ref[1])
    na = na_ref[...]
    dot = lambda a, b: jnp.dot(a, b[...], preferred_element_type=jnp.float32)
    hm_ref[0] = dot(h0, wm00) + dot(h1, wm10)
    hm_ref[1] = dot(h0, wm01) + dot(h1, wm11)
    sc_ref[0] = (dot(h0, ws00) + dot(h1, ws10)) * na
    sc_ref[1] = (dot(h0, ws01) + dot(h1, ws11)) * na


def _node_call(agg, scp, na, wquads):
    hin = agg.shape[2]
    h = wquads[0].shape[1]
    full = lambda shape: pl.BlockSpec(shape, lambda e: tuple(0 for _ in shape))
    return pl.pallas_call(
        _node_body,
        grid=(N // BN,),
        in_specs=[pl.BlockSpec((2, BN, hin), lambda e: (0, e, 0)),
                  pl.BlockSpec((2, BN, hin), lambda e: (0, e, 0)),
                  pl.BlockSpec((BN, 1), lambda e: (e, 0))] +
                 [full(w.shape) for w in wquads],
        out_specs=[pl.BlockSpec((2, BN, h), lambda e: (0, e, 0)),
                   pl.BlockSpec((2, BN, h), lambda e: (0, e, 0))],
        out_shape=[jax.ShapeDtypeStruct((2, N, h), jnp.float32),
                   jax.ShapeDtypeStruct((2, N, h), jnp.float32)],
    )(agg, scp, na, *wquads)


def _make_sc_kernel(h):
    tiledwp = (h == 144)
    nv = h // 16
    mesh = plsc.VectorSubcoreMesh(core_axis_name="c", subcore_axis_name="s")

    if tiledwp:
        wp_scratch = [pltpu.VMEM((CG, 8, 128), jnp.float32),
                      pltpu.VMEM((CG, 8, 128), jnp.float32),
                      pltpu.VMEM((CG, 8, 16), jnp.float32),
                      pltpu.VMEM((CG, 8, 16), jnp.float32),
                      pltpu.VMEM((CG, 8, 128), jnp.float32),
                      pltpu.VMEM((CG, 8, 128), jnp.float32),
                      pltpu.VMEM((CG, 8, 16), jnp.float32),
                      pltpu.VMEM((CG, 8, 16), jnp.float32)]
    else:
        wp_scratch = [pltpu.VMEM((C, h), jnp.float32),
                      pltpu.VMEM((C, h), jnp.float32),
                      pltpu.VMEM((C, h), jnp.float32),
                      pltpu.VMEM((C, h), jnp.float32)]

    def body(hm_a, hm_b, w_hbm, p_hbm, src_hbm, dst_hbm, zeros_hbm, agg_hbm,
             spmem, rows0, rows1, rows2, rows3, *rest):
        wpb = rest[:len(wp_scratch)]
        rest = rest[len(wp_scratch):]
        (srcv0, srcv1, srcv2, srcv3, dstv0, dstv1, dstv2, dstv3,
         sdst0, sdst1, sdst2, sdst3,
         sem_g0, sem_g1, sem_w0, sem_w1,
         sem_s0, sem_s1, sem_s2, sem_s3,
         sem_i0, sem_i1, sem_i2, sem_i3) = rest
        c = lax.axis_index("c")
        s = lax.axis_index("s")
        rowsb = (rows0, rows1, rows2, rows3)
        sdst = (sdst0, sdst1, sdst2, sdst3)
        if tiledwp:
            wbig = (wpb[0], wpb[1])
            wsm = (wpb[2], wpb[3])
            pbig = (wpb[4], wpb[5])
            psm = (wpb[6], wpb[7])
        else:
            wb = (wpb[0], wpb[1])
            pb = (wpb[2], wpb[3])
        srcv = (srcv0, srcv1, srcv2, srcv3)
        dstv = (dstv0, dstv1, dstv2, dstv3)
        sem_g = (sem_g0, sem_g1)
        sem_w = (sem_w0, sem_w1)
        sem_s = (sem_s0, sem_s1, sem_s2, sem_s3)
        sem_i = (sem_i0, sem_i1, sem_i2, sem_i3)

        @pl.when(s < 10)
        def _():
            pltpu.sync_copy(zeros_hbm, spmem.at[pl.ds(s * 1000, 1000)])
        plsc.subcore_barrier()

        base0 = s * EPS

        def issue_idx(k, sl):
            pltpu.async_copy(src_hbm.at[pl.ds(base0 + k * C, C)], srcv[sl],
                             sem_i[sl])
            pltpu.async_copy(dst_hbm.at[pl.ds(base0 + k * C, C)], dstv[sl],
                             sem_i[sl])

        def wait_idx(sl):
            pltpu.make_async_copy(src_hbm.at[pl.ds(0, C)], srcv[sl],
                                  sem_i[sl]).wait()
            pltpu.make_async_copy(src_hbm.at[pl.ds(0, C)], dstv[sl],
                                  sem_i[sl]).wait()

        def run(hm_hbm, core):
            def issue(k, sl, r4, q):
                pltpu.async_copy(hm_hbm.at[srcv[sl]], rowsb[r4], sem_g[q])
                if tiledwp:
                    gb = core * (E // 8) + (base0 + k * C) // 8
                    pltpu.async_copy(w_hbm.at[pl.ds(gb, CG), 0], wbig[q],
                                     sem_w[q])
                    pltpu.async_copy(w_hbm.at[pl.ds(gb, CG), 1, :, pl.ds(0, 16)],
                                     wsm[q], sem_w[q])
                    pltpu.async_copy(p_hbm.at[pl.ds(gb, CG), 0], pbig[q],
                                     sem_w[q])
                    pltpu.async_copy(p_hbm.at[pl.ds(gb, CG), 1, :, pl.ds(0, 16)],
                                     psm[q], sem_w[q])
                else:
                    base = core * E + base0 + k * C
                    pltpu.async_copy(w_hbm.at[pl.ds(base, C)], wpb[0 + q],
                                     sem_w[q])
                    pltpu.async_copy(p_hbm.at[pl.ds(base, C)], wpb[2 + q],
                                     sem_w[q])

            def wait_wp(p):
                if tiledwp:
                    pltpu.make_async_copy(w_hbm.at[pl.ds(0, CG), 0],
                                          wbig[p], sem_w[p]).wait()
                    pltpu.make_async_copy(w_hbm.at[pl.ds(0, CG), 1, :,
                                                   pl.ds(0, 16)],
                                          wsm[p], sem_w[p]).wait()
                    pltpu.make_async_copy(p_hbm.at[pl.ds(0, CG), 0],
                                          pbig[p], sem_w[p]).wait()
                    pltpu.make_async_copy(p_hbm.at[pl.ds(0, CG), 1, :,
                                                   pl.ds(0, 16)],
                                          psm[p], sem_w[p]).wait()
                else:
                    pltpu.make_async_copy(w_hbm.at[pl.ds(0, C)],
                                          wpb[0 + p], sem_w[p]).wait()
                    pltpu.make_async_copy(p_hbm.at[pl.ds(0, C)],
                                          wpb[2 + p], sem_w[p]).wait()

            issue_idx(0, 0)
            issue_idx(1, 1)
            issue_idx(2, 2)
            wait_idx(0)
            issue(0, 0, 0, 0)

            def chunk(k, sl, p):
                @pl.when(k < NCHUNK)
                def _():
                    pltpu.make_async_copy(hm_hbm.at[srcv[sl]], rowsb[sl],
                                          sem_g[p]).wait()
                    wait_wp(p)

                    @pl.when(k + 1 < NCHUNK)
                    def _():
                        @pl.when(k >= 1)
                        def _():
                            pltpu.make_async_copy(hm_a.at[pl.ds(0, C)],
                                                  rowsb[(sl + 3) % 4],
                                                  sem_s[(sl + 3) % 4]).wait()
                        wait_idx((sl + 1) % 4)
                        issue(k + 1, (sl + 1) % 4, (sl + 1) % 4, 1 - p)

                    def edge(i, carry):
                        if tiledwp:
                            g = i // 8
                            r = i % 8
                            for t in range(8):
                                esl = pl.ds(t * 16, 16)
                                rowsb[sl][i, esl] = (rowsb[sl][i, esl]
                                                     * wbig[p][g, r, esl]
                                                     + pbig[p][g, r, esl])
                            esl = pl.ds(128, 16)
                            rowsb[sl][i, esl] = (rowsb[sl][i, esl]
                                                 * wsm[p][g, r, :]
                                                 + psm[p][g, r, :])
                        else:
                            for t in range(nv):
                                esl = pl.ds(t * 16, 16)
                                rowsb[sl][i, esl] = (rowsb[sl][i, esl]
                                                     * wpb[0 + p][i, esl]
                                                     + wpb[2 + p][i, esl])
                        return carry
                    lax.fori_loop(0, C, edge, 0, unroll=2)

                    for t in range(C // 16):
                        tsl = pl.ds(t * 16, 16)
                        sdst[sl][tsl] = dstv[sl][tsl]

                    pltpu.async_copy(rowsb[sl], spmem.at[sdst[sl]],
                                     sem_s[sl], add=True)

                    @pl.when(k + 3 < NCHUNK)
                    def _():
                        issue_idx(k + 3, (sl + 3) % 4)

            def quad(jj, carry):
                for u in range(4):
                    chunk(4 * jj + u, u, u % 2)
                return carry
            lax.fori_loop(0, (NCHUNK + 3) // 4, quad, 0)

            for kk in range(NCHUNK - 2, NCHUNK):
                pltpu.make_async_copy(hm_a.at[pl.ds(0, C)], rowsb[kk % 4],
                                      sem_s[kk % 4]).wait()

        @pl.when(c == 0)
        def _():
            run(hm_a, 0)

        @pl.when(c == 1)
        def _():
            run(hm_b, 1)

        plsc.subcore_barrier()

        @pl.when(s < 10)
        def _():
            pltpu.sync_copy(spmem.at[pl.ds(s * 1000, 1000)],
                            agg_hbm.at[pl.ds(c * N + s * 1000, 1000)])

    return pl.kernel(
        body,
        out_type=jax.ShapeDtypeStruct((2 * N, h), jnp.float32),
        mesh=mesh,
        compiler_params=pltpu.CompilerParams(use_tc_tiling_on_sc=False,
                                             needs_layout_passes=False),
        scratch_types=[
            pltpu.VMEM_SHARED((N, h), jnp.float32),
            pltpu.VMEM((C, h), jnp.float32),
            pltpu.VMEM((C, h), jnp.float32),
            pltpu.VMEM((C, h), jnp.float32),
            pltpu.VMEM((C, h), jnp.float32),
        ] + wp_scratch + [pltpu.VMEM((C,), jnp.int32)] * 12
          + [pltpu.SemaphoreType.DMA] * 12,
    )


def _final_body(agg_ref, scp_ref, batch_ref, fc1w, fc1b, fc2w, fc2b, ow, ob,
                out_ref):
    h0 = agg_ref[0] * INV_SQRT_NN + scp_ref[0]
    h1 = agg_ref[1] * INV_SQRT_NN + scp_ref[1]
    seg = lax.broadcasted_iota(jnp.int32, (G, N), 0)
    mask = (batch_ref[...] == seg).astype(jnp.float32)
    g0 = jnp.dot(mask, h0, preferred_element_type=jnp.float32)
    g1 = jnp.dot(mask, h1, preferred_element_type=jnp.float32)
    g = jnp.concatenate([g0, g1], axis=1)
    g = jax.nn.gelu(jnp.dot(g, fc1w[...], preferred_element_type=jnp.float32) + fc1b[...])
    g = jax.nn.gelu(jnp.dot(g, fc2w[...], preferred_element_type=jnp.float32) + fc2b[...])
    logits = jnp.dot(g, ow[...], preferred_element_type=jnp.float32) + ob[...]
    out_ref[...] = jax.nn.softmax(logits, axis=1)


def _final_call(agg, scp, batch2d, fc1w, fc1b, fc2w, fc2b, ow, ob):
    full = lambda shape: pl.BlockSpec(shape, lambda: tuple(0 for _ in shape))
    args = (agg, scp, batch2d, fc1w, fc1b, fc2w, fc2b, ow, ob)
    return pl.pallas_call(
        _final_body,
        in_specs=[full(a.shape) for a in args],
        out_specs=full((G, 16)),
        out_shape=jax.ShapeDtypeStruct((G, 16), jnp.float32),
    )(*args)


def kernel(x, node_attr, edge_src, edge_dst, edge_attr, edge_length_embedding,
           batch, params):
    halves = [d // 2 for d in DOUTS]

    na = node_attr
    src = edge_src.astype(jnp.int32)
    dst = edge_dst.astype(jnp.int32)

    wps = []
    for l in range(4):
        h = halves[l]
        f3 = params[f'F3_{l}']
        we = params[f'Wedge{l}']
        wl = [params[f'F1_{l}'], params[f'F2_{l}'],
              f3[:, :h], f3[:, h:], we[:, :h], we[:, h:]]
        if h == 144:
            wps.append(_radial_call144(edge_length_embedding, edge_attr, wl))
        else:
            wps.append(_radial_call32(edge_length_embedding, edge_attr, wl))

    wm, ws = params['Wmsg0'], params['Wsc0']
    h0 = halves[0]
    hm, sc = _node0_call(x, na, wm[:, :h0], wm[:, h0:], ws[:, :h0], ws[:, h0:])

    agg = None
    for l in range(4):
        h = halves[l]
        w_st, p_st = wps[l]
        if h == 144:
            w_in = w_st.reshape(2 * (E // 8), 2, 8, 128)
            p_in = p_st.reshape(2 * (E // 8), 2, 8, 128)
        else:
            w_in = w_st.reshape(2 * E, h)
            p_in = p_st.reshape(2 * E, h)
        zeros = jnp.zeros((1000, h), jnp.float32)
        sck = _make_sc_kernel(h)
        agg = sck(hm[0], hm[1], w_in, p_in, src, dst, zeros)
        if l < 3:
            hn = halves[l + 1]
            hin = h
            wm, ws = params[f'Wmsg{l + 1}'], params[f'Wsc{l + 1}']
            quads = [wm[:hin, :hn], wm[:hin, hn:], wm[hin:, :hn], wm[hin:, hn:],
                     ws[:hin, :hn], ws[:hin, hn:], ws[hin:, :hn], ws[hin:, hn:]]
            hm, sc = _node_call(agg.reshape(2, N, h), sc, na, quads)

    p = params
    return _final_call(agg.reshape(2, N, halves[3]), sc, batch.reshape(1, N),
                       p['fc1_w'], p['fc1_b'].reshape(1, -1),
                       p['fc2_w'], p['fc2_b'].reshape(1, -1),
                       p['out_w'], p['out_b'].reshape(1, -1))


# speedup vs baseline: 1.7403x; 1.0451x over previous
import functools

import jax
import jax.numpy as jnp
import numpy as np
from jax import lax
from jax.experimental import pallas as pl
from jax.experimental.pallas import tpu as pltpu
from jax.experimental.pallas import tpu_sc as plsc

N = 10000
E = 320000
G = 64
NUM_NEIGHBORS = 32.0
INV_SQRT_NN = 1.0 / (NUM_NEIGHBORS ** 0.5)
DOUTS = [288, 288, 288, 64]

BE = 1280
BN = 1000
NSUB = 16
EPS = E // NSUB
C = 32
CG = C // 8
NCHUNK = EPS // C


def _radial_body144(elem_ref, ea_ref, f1, f2, f3a, f3b, wea, web, w_ref, p_ref):
    elem = elem_ref[...]
    ea = ea_ref[...]
    w2 = jax.nn.gelu(jnp.dot(elem, f1[...], preferred_element_type=jnp.float32))
    w2 = jax.nn.gelu(jnp.dot(w2, f2[...], preferred_element_type=jnp.float32))
    wa = jnp.dot(w2, f3a[...], preferred_element_type=jnp.float32)
    wb = jnp.dot(w2, f3b[...], preferred_element_type=jnp.float32)
    ema = jnp.dot(ea, wea[...], preferred_element_type=jnp.float32)
    emb = jnp.dot(ea, web[...], preferred_element_type=jnp.float32)
    pa = ema * wa
    pb = emb * wb
    ng = BE // 8
    for ref, va, vb in ((w_ref, wa, wb), (p_ref, pa, pb)):
        ref[0, :, 0] = va[:, :128].reshape(ng, 8, 128)
        ref[0, :, 1, :, :16] = va[:, 128:].reshape(ng, 8, 16)
        ref[1, :, 0] = vb[:, :128].reshape(ng, 8, 128)
        ref[1, :, 1, :, :16] = vb[:, 128:].reshape(ng, 8, 16)


def _radial_call144(elem, ea, wl):
    full = lambda shape: pl.BlockSpec(shape, lambda e: tuple(0 for _ in shape))
    in_specs = [pl.BlockSpec((BE, 10), lambda e: (e, 0)),
                pl.BlockSpec((BE, 9), lambda e: (e, 0))] + \
               [full(w.shape) for w in wl]
    ospec = pl.BlockSpec((2, BE // 8, 2, 8, 128), lambda e: (0, e, 0, 0, 0))
    oshape = jax.ShapeDtypeStruct((2, E // 8, 2, 8, 128), jnp.float32)
    return pl.pallas_call(
        _radial_body144,
        grid=(E // BE,),
        in_specs=in_specs,
        out_specs=[ospec, ospec],
        out_shape=[oshape, oshape],
    )(elem, ea, *wl)


def _radial_body32(elem_ref, ea_ref, f1, f2, f3a, f3b, wea, web, w_ref, p_ref):
    elem = elem_ref[...]
    ea = ea_ref[...]
    w2 = jax.nn.gelu(jnp.dot(elem, f1[...], preferred_element_type=jnp.float32))
    w2 = jax.nn.gelu(jnp.dot(w2, f2[...], preferred_element_type=jnp.float32))
    wa = jnp.dot(w2, f3a[...], preferred_element_type=jnp.float32)
    wb = jnp.dot(w2, f3b[...], preferred_element_type=jnp.float32)
    ema = jnp.dot(ea, wea[...], preferred_element_type=jnp.float32)
    emb = jnp.dot(ea, web[...], preferred_element_type=jnp.float32)
    w_ref[0] = wa
    w_ref[1] = wb
    p_ref[0] = ema * wa
    p_ref[1] = emb * wb


def _radial_call32(elem, ea, wl):
    h = wl[2].shape[1]
    full = lambda shape: pl.BlockSpec(shape, lambda e: tuple(0 for _ in shape))
    in_specs = [pl.BlockSpec((BE, 10), lambda e: (e, 0)),
                pl.BlockSpec((BE, 9), lambda e: (e, 0))] + \
               [full(w.shape) for w in wl]
    return pl.pallas_call(
        _radial_body32,
        grid=(E // BE,),
        in_specs=in_specs,
        out_specs=[pl.BlockSpec((2, BE, h), lambda e: (0, e, 0)),
                   pl.BlockSpec((2, BE, h), lambda e: (0, e, 0))],
        out_shape=[jax.ShapeDtypeStruct((2, E, h), jnp.float32),
                   jax.ShapeDtypeStruct((2, E, h), jnp.float32)],
    )(elem, ea, *wl)


def _node0_body(x_ref, na_ref, wma, wmb, wsa, wsb, hm_ref, sc_ref):
    x = x_ref[...]
    na = na_ref[...]
    hm_ref[0] = jnp.dot(x, wma[...], preferred_element_type=jnp.float32)
    hm_ref[1] = jnp.dot(x, wmb[...], preferred_element_type=jnp.float32)
    sc_ref[0] = jnp.dot(x, wsa[...], preferred_element_type=jnp.float32) * na
    sc_ref[1] = jnp.dot(x, wsb[...], preferred_element_type=jnp.float32) * na


def _node0_call(x, na, wma, wmb, wsa, wsb):
    h = wma.shape[1]
    full = lambda shape: pl.BlockSpec(shape, lambda e: tuple(0 for _ in shape))
    return pl.pallas_call(
        _node0_body,
        grid=(N // BN,),
        in_specs=[pl.BlockSpec((BN, 128), lambda e: (e, 0)),
                  pl.BlockSpec((BN, 1), lambda e: (e, 0)),
                  full(wma.shape), full(wmb.shape), full(wsa.shape), full(wsb.shape)],
        out_specs=[pl.BlockSpec((2, BN, h), lambda e: (0, e, 0)),
                   pl.BlockSpec((2, BN, h), lambda e: (0, e, 0))],
        out_shape=[jax.ShapeDtypeStruct((2, N, h), jnp.float32),
                   jax.ShapeDtypeStruct((2, N, h), jnp.float32)],
    )(x, na, wma, wmb, wsa, wsb)


def _node_body(agg_ref, scp_ref, na_ref, wm00, wm01, wm10, wm11,
               ws00, ws01, ws10, ws11, hm_ref, sc_ref):
    h0 = jax.nn.gelu(agg_ref[0] * INV_SQRT_NN + scp_ref[0])
    h1 = jax.nn.gelu(agg_ref[1] * INV_SQRT_NN + scp_ref[1])
    na = na_ref[...]
    dot = lambda a, b: jnp.dot(a, b[...], preferred_element_type=jnp.float32)
    hm_ref[0] = dot(h0, wm00) + dot(h1, wm10)
    hm_ref[1] = dot(h0, wm01) + dot(h1, wm11)
    sc_ref[0] = (dot(h0, ws00) + dot(h1, ws10)) * na
    sc_ref[1] = (dot(h0, ws01) + dot(h1, ws11)) * na


def _node_call(agg, scp, na, wquads):
    hin = agg.shape[2]
    h = wquads[0].shape[1]
    full = lambda shape: pl.BlockSpec(shape, lambda e: tuple(0 for _ in shape))
    return pl.pallas_call(
        _node_body,
        grid=(N // BN,),
        in_specs=[pl.BlockSpec((2, BN, hin), lambda e: (0, e, 0)),
                  pl.BlockSpec((2, BN, hin), lambda e: (0, e, 0)),
                  pl.BlockSpec((BN, 1), lambda e: (e, 0))] +
                 [full(w.shape) for w in wquads],
        out_specs=[pl.BlockSpec((2, BN, h), lambda e: (0, e, 0)),
                   pl.BlockSpec((2, BN, h), lambda e: (0, e, 0))],
        out_shape=[jax.ShapeDtypeStruct((2, N, h), jnp.float32),
                   jax.ShapeDtypeStruct((2, N, h), jnp.float32)],
    )(agg, scp, na, *wquads)


def _make_sc_kernel(h):
    tiledwp = (h == 144)
    nv = h // 16
    mesh = plsc.VectorSubcoreMesh(core_axis_name="c", subcore_axis_name="s")

    if tiledwp:
        wp_scratch = [pltpu.VMEM((CG, 8, 128), jnp.float32),
                      pltpu.VMEM((CG, 8, 128), jnp.float32),
                      pltpu.VMEM((CG, 8, 16), jnp.float32),
                      pltpu.VMEM((CG, 8, 16), jnp.float32),
                      pltpu.VMEM((CG, 8, 128), jnp.float32),
                      pltpu.VMEM((CG, 8, 128), jnp.float32),
                      pltpu.VMEM((CG, 8, 16), jnp.float32),
                      pltpu.VMEM((CG, 8, 16), jnp.float32)]
    else:
        wp_scratch = [pltpu.VMEM((C, h), jnp.float32),
                      pltpu.VMEM((C, h), jnp.float32),
                      pltpu.VMEM((C, h), jnp.float32),
                      pltpu.VMEM((C, h), jnp.float32)]

    def body(hm_a, hm_b, w_hbm, p_hbm, src_hbm, dst_hbm, zeros_hbm, agg_hbm,
             spmem, rows0, rows1, rows2, rows3, *rest):
        wpb = rest[:len(wp_scratch)]
        rest = rest[len(wp_scratch):]
        (srcv0, srcv1, srcv2, srcv3, dstv0, dstv1, dstv2, dstv3,
         sdst0, sdst1, sdst2, sdst3,
         sem_g0, sem_g1, sem_w0, sem_w1,
         sem_s0, sem_s1, sem_s2, sem_s3,
         sem_i0, sem_i1, sem_i2, sem_i3) = rest
        c = lax.axis_index("c")
        s = lax.axis_index("s")
        rowsb = (rows0, rows1, rows2, rows3)
        sdst = (sdst0, sdst1, sdst2, sdst3)
        if tiledwp:
            wbig = (wpb[0], wpb[1])
            wsm = (wpb[2], wpb[3])
            pbig = (wpb[4], wpb[5])
            psm = (wpb[6], wpb[7])
        else:
            wb = (wpb[0], wpb[1])
            pb = (wpb[2], wpb[3])
        srcv = (srcv0, srcv1, srcv2, srcv3)
        dstv = (dstv0, dstv1, dstv2, dstv3)
        sem_g = (sem_g0, sem_g1)
        sem_w = (sem_w0, sem_w1)
        sem_s = (sem_s0, sem_s1, sem_s2, sem_s3)
        sem_i = (sem_i0, sem_i1, sem_i2, sem_i3)

        @pl.when(s < 10)
        def _():
            pltpu.sync_copy(zeros_hbm, spmem.at[pl.ds(s * 1000, 1000)])
        plsc.subcore_barrier()

        base0 = s * EPS

        def issue_idx(k, sl):
            pltpu.async_copy(src_hbm.at[pl.ds(base0 + k * C, C)], srcv[sl],
                             sem_i[sl])
            pltpu.async_copy(dst_hbm.at[pl.ds(base0 + k * C, C)], dstv[sl],
                             sem_i[sl])

        def wait_idx(sl):
            pltpu.make_async_copy(src_hbm.at[pl.ds(0, C)], srcv[sl],
                                  sem_i[sl]).wait()
            pltpu.make_async_copy(src_hbm.at[pl.ds(0, C)], dstv[sl],
                                  sem_i[sl]).wait()

        def run(hm_hbm, core):
            def issue(k, sl, r4, q):
                pltpu.async_copy(hm_hbm.at[srcv[sl]], rowsb[r4], sem_g[q])
                if tiledwp:
                    gb = core * (E // 8) + (base0 + k * C) // 8
                    pltpu.async_copy(w_hbm.at[pl.ds(gb, CG), 0], wbig[q],
                                     sem_w[q])
                    pltpu.async_copy(w_hbm.at[pl.ds(gb, CG), 1, :, pl.ds(0, 16)],
                                     wsm[q], sem_w[q])
                    pltpu.async_copy(p_hbm.at[pl.ds(gb, CG), 0], pbig[q],
                                     sem_w[q])
                    pltpu.async_copy(p_hbm.at[pl.ds(gb, CG), 1, :, pl.ds(0, 16)],
                                     psm[q], sem_w[q])
                else:
                    base = core * E + base0 + k * C
                    pltpu.async_copy(w_hbm.at[pl.ds(base, C)], wpb[0 + q],
                                     sem_w[q])
                    pltpu.async_copy(p_hbm.at[pl.ds(base, C)], wpb[2 + q],
                                     sem_w[q])

            def wait_wp(p):
                if tiledwp:
                    pltpu.make_async_copy(w_hbm.at[pl.ds(0, CG), 0],
                                          wbig[p], sem_w[p]).wait()
                    pltpu.make_async_copy(w_hbm.at[pl.ds(0, CG), 1, :,
                                                   pl.ds(0, 16)],
                                          wsm[p], sem_w[p]).wait()
                    pltpu.make_async_copy(p_hbm.at[pl.ds(0, CG), 0],
                                          pbig[p], sem_w[p]).wait()
                    pltpu.make_async_copy(p_hbm.at[pl.ds(0, CG), 1, :,
                                                   pl.ds(0, 16)],
                                          psm[p], sem_w[p]).wait()
                else:
                    pltpu.make_async_copy(w_hbm.at[pl.ds(0, C)],
                                          wpb[0 + p], sem_w[p]).wait()
                    pltpu.make_async_copy(p_hbm.at[pl.ds(0, C)],
                                          wpb[2 + p], sem_w[p]).wait()

            issue_idx(0, 0)
            issue_idx(1, 1)
            issue_idx(2, 2)
            wait_idx(0)
            issue(0, 0, 0, 0)

            def chunk(k, sl, p):
                @pl.when(k < NCHUNK)
                def _():
                    pltpu.make_async_copy(hm_hbm.at[srcv[sl]], rowsb[sl],
                                          sem_g[p]).wait()
                    wait_wp(p)

                    @pl.when(k + 1 < NCHUNK)
                    def _():
                        wait_idx((sl + 1) % 4)
                        issue(k + 1, (sl + 1) % 4, (sl + 1) % 4, 1 - p)

                    def edge(i, carry):
                        if tiledwp:
                            g = i // 8
                            r = i % 8
                            for t in range(8):
                                esl = pl.ds(t * 16, 16)
                                rowsb[sl][i, esl] = (rowsb[sl][i, esl]
                                                     * wbig[p][g, r, esl]
                                                     + pbig[p][g, r, esl])
                            esl = pl.ds(128, 16)
                            rowsb[sl][i, esl] = (rowsb[sl][i, esl]
                                                 * wsm[p][g, r, :]
                                                 + psm[p][g, r, :])
                        else:
                            for t in range(nv):
                                esl = pl.ds(t * 16, 16)
                                rowsb[sl][i, esl] = (rowsb[sl][i, esl]
                                                     * wpb[0 + p][i, esl]
                                                     + wpb[2 + p][i, esl])
                        return carry
                    lax.fori_loop(0, C, edge, 0, unroll=2)

                    for t in range(C // 16):
                        tsl = pl.ds(t * 16, 16)
                        sdst[sl][tsl] = dstv[sl][tsl]

                    @pl.when(k >= 1)
                    def _():
                        pltpu.make_async_copy(hm_a.at[pl.ds(0, C)],
                                              rowsb[(sl + 3) % 4],
                                              sem_s[(sl + 3) % 4]).wait()
                    pltpu.async_copy(rowsb[sl], spmem.at[sdst[sl]],
                                     sem_s[sl], add=True)

                    @pl.when(k + 3 < NCHUNK)
                    def _():
                        issue_idx(k + 3, (sl + 3) % 4)

            def quad(jj, carry):
                for u in range(4):
                    chunk(4 * jj + u, u, u % 2)
                return carry
            lax.fori_loop(0, (NCHUNK + 3) // 4, quad, 0)

            for kk in range(NCHUNK - 1, NCHUNK):
                pltpu.make_async_copy(hm_a.at[pl.ds(0, C)], rowsb[kk % 4],
                                      sem_s[kk % 4]).wait()

        @pl.when(c == 0)
        def _():
            run(hm_a, 0)

        @pl.when(c == 1)
        def _():
            run(hm_b, 1)

        plsc.subcore_barrier()

        @pl.when(s < 10)
        def _():
            pltpu.sync_copy(spmem.at[pl.ds(s * 1000, 1000)],
                            agg_hbm.at[pl.ds(c * N + s * 1000, 1000)])

    return pl.kernel(
        body,
        out_type=jax.ShapeDtypeStruct((2 * N, h), jnp.float32),
        mesh=mesh,
        compiler_params=pltpu.CompilerParams(use_tc_tiling_on_sc=False,
                                             needs_layout_passes=False),
        scratch_types=[
            pltpu.VMEM_SHARED((N, h), jnp.float32),
            pltpu.VMEM((C, h), jnp.float32),
            pltpu.VMEM((C, h), jnp.float32),
            pltpu.VMEM((C, h), jnp.float32),
            pltpu.VMEM((C, h), jnp.float32),
        ] + wp_scratch + [pltpu.VMEM((C,), jnp.int32)] * 12
          + [pltpu.SemaphoreType.DMA] * 12,
    )


def _final_body(agg_ref, scp_ref, batch_ref, fc1w, fc1b, fc2w, fc2b, ow, ob,
                out_ref):
    h0 = agg_ref[0] * INV_SQRT_NN + scp_ref[0]
    h1 = agg_ref[1] * INV_SQRT_NN + scp_ref[1]
    seg = lax.broadcasted_iota(jnp.int32, (G, N), 0)
    mask = (batch_ref[...] == seg).astype(jnp.float32)
    g0 = jnp.dot(mask, h0, preferred_element_type=jnp.float32)
    g1 = jnp.dot(mask, h1, preferred_element_type=jnp.float32)
    g = jnp.concatenate([g0, g1], axis=1)
    g = jax.nn.gelu(jnp.dot(g, fc1w[...], preferred_element_type=jnp.float32) + fc1b[...])
    g = jax.nn.gelu(jnp.dot(g, fc2w[...], preferred_element_type=jnp.float32) + fc2b[...])
    logits = jnp.dot(g, ow[...], preferred_element_type=jnp.float32) + ob[...]
    out_ref[...] = jax.nn.softmax(logits, axis=1)


def _final_call(agg, scp, batch2d, fc1w, fc1b, fc2w, fc2b, ow, ob):
    full = lambda shape: pl.BlockSpec(shape, lambda: tuple(0 for _ in shape))
    args = (agg, scp, batch2d, fc1w, fc1b, fc2w, fc2b, ow, ob)
    return pl.pallas_call(
        _final_body,
        in_specs=[full(a.shape) for a in args],
        out_specs=full((G, 16)),
        out_shape=jax.ShapeDtypeStruct((G, 16), jnp.float32),
    )(*args)


def kernel(x, node_attr, edge_src, edge_dst, edge_attr, edge_length_embedding,
           batch, params):
    halves = [d // 2 for d in DOUTS]

    na = node_attr
    src = edge_src.astype(jnp.int32)
    dst = edge_dst.astype(jnp.int32)

    wps = []
    for l in range(4):
        h = halves[l]
        f3 = params[f'F3_{l}']
        we = params[f'Wedge{l}']
        wl = [params[f'F1_{l}'], params[f'F2_{l}'],
              f3[:, :h], f3[:, h:], we[:, :h], we[:, h:]]
        if h == 144:
            wps.append(_radial_call144(edge_length_embedding, edge_attr, wl))
        else:
            wps.append(_radial_call32(edge_length_embedding, edge_attr, wl))

    wm, ws = params['Wmsg0'], params['Wsc0']
    h0 = halves[0]
    hm, sc = _node0_call(x, na, wm[:, :h0], wm[:, h0:], ws[:, :h0], ws[:, h0:])

    agg = None
    for l in range(4):
        h = halves[l]
        w_st, p_st = wps[l]
        if h == 144:
            w_in = w_st.reshape(2 * (E // 8), 2, 8, 128)
            p_in = p_st.reshape(2 * (E // 8), 2, 8, 128)
        else:
            w_in = w_st.reshape(2 * E, h)
            p_in = p_st.reshape(2 * E, h)
        zeros = jnp.zeros((1000, h), jnp.float32)
        sck = _make_sc_kernel(h)
        agg = sck(hm[0], hm[1], w_in, p_in, src, dst, zeros)
        if l < 3:
            hn = halves[l + 1]
            hin = h
            wm, ws = params[f'Wmsg{l + 1}'], params[f'Wsc{l + 1}']
            quads = [wm[:hin, :hn], wm[:hin, hn:], wm[hin:, :hn], wm[hin:, hn:],
                     ws[:hin, :hn], ws[:hin, hn:], ws[hin:, :hn], ws[hin:, hn:]]
            hm, sc = _node_call(agg.reshape(2, N, h), sc, na, quads)

    p = params
    return _final_call(agg.reshape(2, N, halves[3]), sc, batch.reshape(1, N),
                       p['fc1_w'], p['fc1_b'].reshape(1, -1),
                       p['fc2_w'], p['fc2_b'].reshape(1, -1),
                       p['out_w'], p['out_b'].reshape(1, -1))
